# Initial kernel scaffold; baseline (speedup 1.0000x reference)
import jax
import jax.numpy as jnp
from jax.experimental import pallas as pl


def kernel(input_ids, token_embeddings, elo_weak, elo_strong):
    raise NotImplementedError("write your pallas kernel here")


# speedup vs baseline: 2.1586x; 2.1586x over previous
import functools

import jax
import jax.numpy as jnp
from jax import lax
from jax.experimental import pallas as pl
from jax.experimental.pallas import tpu as pltpu
from jax.experimental.pallas import tpu_sc as plsc

VOCAB = 100000
ELO_MIN = 500.0
ELO_MAX = 3000.0

L = 16
NC, NS = 2, 16
NW = NC * NS

B, S, D = 4096, 200, 64
N = B * S
PER_W = N // NW
CHUNK = 1024
NCHUNK = PER_W // CHUNK


def _body(ids_hbm, table_hbm, weak_hbm, strong_hbm, out_hbm,
          idx_raw, idx_safe, posb, gamb, rows, wpad, wbcast, sbcast, sem):
    wid = lax.axis_index("s") * NC + lax.axis_index("c")
    base = wid * PER_W

    pltpu.sync_copy(weak_hbm, wpad.at[pl.ds(L, D)])
    for d in range(D):
        wbcast[d, :] = plsc.load_gather(wpad, [jnp.full((L,), L + d, jnp.int32)])
    pltpu.sync_copy(strong_hbm, wpad.at[pl.ds(L, D)])
    for d in range(D):
        sbcast[d, :] = plsc.load_gather(wpad, [jnp.full((L,), L + d, jnp.int32)])

    lanes = lax.iota(jnp.int32, L)

    def chunk_body(ci, _):
        cbase = base + ci * CHUNK
        pltpu.sync_copy(ids_hbm.at[pl.ds(cbase, CHUNK)], idx_raw)

        def vreg_body(i, cnt):
            v = idx_raw[pl.ds(i * L, L)]
            m = v >= VOCAB
            safe = jnp.where(m, 0, v)
            idx_safe[pl.ds(i * L, L)] = safe
            mi = m.astype(jnp.int32)
            pref = plsc.cumsum(mi)
            tot = jnp.sum(mi)
            dst = jnp.maximum(cnt + pref - 1, 0)
            elo_f = (v - VOCAB).astype(jnp.float32)
            gam = jnp.clip((ELO_MAX - elo_f) / (ELO_MAX - ELO_MIN), 0.0, 1.0)
            pos = i * L + lanes
            plsc.store_scatter(posb, [dst], pos, mask=m)
            plsc.store_scatter(gamb, [dst], gam, mask=m)
            return cnt + tot

        cnt = lax.fori_loop(0, CHUNK // L, vreg_body, jnp.int32(0))

        pltpu.async_copy(table_hbm.at[idx_safe], rows, sem).wait()

        ngrp = (cnt + (L - 1)) // L

        def fix_body(g, _):
            goff = g * L
            pos = posb[pl.ds(goff, L)]
            gam = gamb[pl.ds(goff, L)]
            valid = (goff + lanes) < cnt
            omg = 1.0 - gam
            for d in range(D):
                dd = jnp.full((L,), d, dtype=jnp.int32)
                wv = wbcast[d, :]
                sv = sbcast[d, :]
                val = gam * wv + omg * sv
                plsc.store_scatter(rows, [pos, dd], val, mask=valid)
            return 0

        lax.fori_loop(0, ngrp, fix_body, 0)

        pltpu.sync_copy(rows, out_hbm.at[pl.ds(cbase, CHUNK)])
        return 0

    lax.fori_loop(0, NCHUNK, chunk_body, 0)


_SCRATCH = [
    pltpu.VMEM((CHUNK,), jnp.int32),
    pltpu.VMEM((CHUNK,), jnp.int32),
    pltpu.VMEM((CHUNK,), jnp.int32),
    pltpu.VMEM((CHUNK,), jnp.float32),
    pltpu.VMEM((CHUNK, D), jnp.float32),
    pltpu.VMEM((L + D,), jnp.float32),
    pltpu.VMEM((D, L), jnp.float32),
    pltpu.VMEM((D, L), jnp.float32),
    pltpu.SemaphoreType.DMA,
]

_emb = pl.kernel(
    _body,
    out_type=jax.ShapeDtypeStruct((N, D), jnp.float32),
    mesh=plsc.VectorSubcoreMesh(
        core_axis_name="c", subcore_axis_name="s",
        num_cores=NC, num_subcores=NS,
    ),
    scratch_types=_SCRATCH,
    compiler_params=pltpu.CompilerParams(
        use_tc_tiling_on_sc=False, needs_layout_passes=False,
    ),
)


def kernel(input_ids, token_embeddings, elo_weak, elo_strong):
    ids = input_ids.reshape(N)
    out = _emb(ids, token_embeddings, elo_weak.reshape(D), elo_strong.reshape(D))
    return out.reshape(input_ids.shape + (D,))
